# baseline (speedup 1.0000x reference)
import jax
import jax.numpy as jnp
from jax import lax
from jax.experimental import pallas as pl
from jax.experimental.pallas import tpu as pltpu
from jax.experimental.pallas import tpu_sc as plsc

NUM_FIELDS = 26
VOCAB = 100000
DIM = 16
BATCH = 4096

NC = 2
NS = 16
L = 16
NW = NC * NS

R = NUM_FIELDS * DIM
TR = R // 8
R_W = R // NW
HALF_A = 50048
HALF_B = 49920
MAIN = HALF_A + HALF_B
TAIL = VOCAB - MAIN
GROUPS = BATCH // L


def _row_body(idx_hbm, tab_hbm, tail_hbm, out_hbm,
              idx_v, bufa_v, bufb_v, tail_v, out_v, sema, semb):
    wid = lax.axis_index("s") * NC + lax.axis_index("c")
    lo_row = wid * R_W
    pltpu.sync_copy(tail_hbm, tail_v)

    def extract_a():
        def do_group(g, carry2):
            iv = idx_v[pl.ds(g * L, L)]
            m = iv < HALF_A
            out_v[pl.ds(g * L, L)] = plsc.load_gather(bufa_v, [iv], mask=m)
            return carry2

        lax.fori_loop(0, GROUPS, do_group, 0)

    def extract_b(rbase):
        def do_group(g, carry2):
            iv = idx_v[pl.ds(g * L, L)]
            mb = (iv >= HALF_A) & (iv < MAIN)
            mt = iv >= MAIN
            gv = plsc.load_gather(bufb_v, [iv - HALF_A], mask=mb)
            tv = plsc.load_gather(tail_v, [iv - MAIN + rbase], mask=mt)
            prev = out_v[pl.ds(g * L, L)]
            out_v[pl.ds(g * L, L)] = jnp.where(mb, gv, jnp.where(mt, tv, prev))
            return carry2

        lax.fori_loop(0, GROUPS, do_group, 0)

    for d in range(8):
        t_lo = (lo_row + 7 - d) // 8
        t_hi = (lo_row + R_W + 7 - d) // 8

        def do_row(t, carry, d=d):
            r = t * 8 + d
            f = r // DIM
            cpa = pltpu.make_async_copy(
                tab_hbm.at[t, d, pl.ds(0, HALF_A)], bufa_v, sema)
            cpb = pltpu.make_async_copy(
                tab_hbm.at[t, d, pl.ds(HALF_A, HALF_B)], bufb_v, semb)
            cpa.start()
            cpb.start()
            pltpu.sync_copy(idx_hbm.at[f, :], idx_v)
            cpa.wait()
            extract_a()
            cpb.wait()
            extract_b(r * TAIL)
            pltpu.sync_copy(out_v, out_hbm.at[t, d, :])
            return carry

        lax.fori_loop(t_lo, t_hi, do_row, 0)


@jax.jit
def _embed_t(idx_t, tab3, tail1):
    mesh = plsc.VectorSubcoreMesh(
        core_axis_name="c", subcore_axis_name="s", num_cores=NC, num_subcores=NS
    )
    return pl.kernel(
        _row_body,
        out_type=jax.ShapeDtypeStruct((TR, 8, BATCH), jnp.float32),
        mesh=mesh,
        scratch_types=[
            pltpu.VMEM((BATCH,), jnp.int32),
            pltpu.VMEM((HALF_A,), jnp.float32),
            pltpu.VMEM((HALF_B,), jnp.float32),
            pltpu.VMEM((R * TAIL,), jnp.float32),
            pltpu.VMEM((BATCH,), jnp.float32),
            pltpu.SemaphoreType.DMA,
            pltpu.SemaphoreType.DMA,
        ],
        compiler_params=pltpu.CompilerParams(
            use_tc_tiling_on_sc=True, needs_layout_passes=False
        ),
    )(idx_t, tab3, tail1)


def kernel(indices, tables):
    idx_t = indices.T
    tab3 = jnp.transpose(tables, (0, 2, 1)).reshape(TR, 8, VOCAB)
    tail1 = jnp.transpose(tables[:, MAIN:, :], (0, 2, 1)).reshape(R * TAIL)
    out_t = _embed_t(idx_t, tab3, tail1)
    return out_t.reshape(R, BATCH).T


# speedup vs baseline: 1.1105x; 1.1105x over previous
import jax
import jax.numpy as jnp
from jax import lax
from jax.experimental import pallas as pl
from jax.experimental.pallas import tpu as pltpu
from jax.experimental.pallas import tpu_sc as plsc

NUM_FIELDS = 26
VOCAB = 100000
DIM = 16
BATCH = 4096

NC = 2
NS = 16
L = 16
NW = NC * NS

R = NUM_FIELDS * DIM
TR = R // 8
R_W = R // NW
MAIN = 99968
TAIL = VOCAB - MAIN
GROUPS = BATCH // L


def _row_body(idx_hbm, tab_hbm, tail_hbm, out_hbm, idx_v, row_v, tail_v, out_v):
    wid = lax.axis_index("s") * NC + lax.axis_index("c")
    lo_row = wid * R_W
    pltpu.sync_copy(tail_hbm, tail_v)

    def extract():
        def do_group(g, carry2):
            iv = idx_v[pl.ds(g * L, L)]
            out_v[pl.ds(g * L, L)] = plsc.load_gather(row_v, [iv])
            return carry2

        lax.fori_loop(0, GROUPS, do_group, 0)

    for d in range(8):
        t_lo = (lo_row + 7 - d) // 8
        t_hi = (lo_row + R_W + 7 - d) // 8

        def do_row(t, carry, d=d):
            r = t * 8 + d
            f = r // DIM
            pltpu.sync_copy(idx_hbm.at[f, :], idx_v)
            pltpu.sync_copy(tab_hbm.at[t, d, pl.ds(0, MAIN)], row_v.at[pl.ds(0, MAIN)])
            rbase = r * TAIL
            for h in range(0, TAIL, L):
                tvals = plsc.load_gather(
                    tail_v, [rbase + h + lax.iota(jnp.int32, L)])
                plsc.store_scatter(
                    row_v, [MAIN + h + lax.iota(jnp.int32, L)], tvals)
            extract()
            pltpu.sync_copy(out_v, out_hbm.at[t, d, :])
            return carry

        lax.fori_loop(t_lo, t_hi, do_row, 0)


@jax.jit
def _embed_t(idx_t, tab3, tail1):
    mesh = plsc.VectorSubcoreMesh(
        core_axis_name="c", subcore_axis_name="s", num_cores=NC, num_subcores=NS
    )
    return pl.kernel(
        _row_body,
        out_type=jax.ShapeDtypeStruct((TR, 8, BATCH), jnp.float32),
        mesh=mesh,
        scratch_types=[
            pltpu.VMEM((BATCH,), jnp.int32),
            pltpu.VMEM((VOCAB,), jnp.float32),
            pltpu.VMEM((R * TAIL,), jnp.float32),
            pltpu.VMEM((BATCH,), jnp.float32),
        ],
        compiler_params=pltpu.CompilerParams(
            use_tc_tiling_on_sc=True, needs_layout_passes=False
        ),
    )(idx_t, tab3, tail1)


def kernel(indices, tables):
    idx_t = indices.T
    tab3 = jnp.transpose(tables, (0, 2, 1)).reshape(TR, 8, VOCAB)
    tail1 = jnp.transpose(tables[:, MAIN:, :], (0, 2, 1)).reshape(R * TAIL)
    out_t = _embed_t(idx_t, tab3, tail1)
    return out_t.reshape(R, BATCH).T


# speedup vs baseline: 1.1243x; 1.0125x over previous
import jax
import jax.numpy as jnp
from jax import lax
from jax.experimental import pallas as pl
from jax.experimental.pallas import tpu as pltpu
from jax.experimental.pallas import tpu_sc as plsc

NUM_FIELDS = 26
VOCAB = 100000
DIM = 16
BATCH = 4096

NC = 2
NS = 16
L = 16
NW = NC * NS

R = NUM_FIELDS * DIM
TR = R // 8
R_W = R // NW
MAIN = 99968
TAIL = VOCAB - MAIN
GROUPS = BATCH // L


def _row_body(idx_hbm, tab_hbm, tail_hbm, out_hbm,
              idx_v, row_v, tail_v, out_v, semr, semo):
    wid = lax.axis_index("s") * NC + lax.axis_index("c")
    lo_row = wid * R_W
    f0 = lo_row // DIM
    f1 = jnp.minimum(f0 + 1, NUM_FIELDS - 1)
    pltpu.sync_copy(tail_hbm, tail_v)
    pltpu.sync_copy(idx_hbm.at[f0, :], idx_v.at[pl.ds(0, BATCH)])
    pltpu.sync_copy(idx_hbm.at[f1, :], idx_v.at[pl.ds(BATCH, BATCH)])

    def extract(ibase, obase):
        def do_group(g, carry2):
            iv = idx_v[pl.ds(ibase + g * L, L)]
            out_v[pl.ds(obase + g * L, L)] = plsc.load_gather(row_v, [iv])
            return carry2

        lax.fori_loop(0, GROUPS, do_group, 0)

    seq0 = 0
    for d in range(8):
        t_lo = (lo_row + 7 - d) // 8
        t_hi = (lo_row + R_W + 7 - d) // 8

        def do_row(t, seq, d=d):
            r = t * 8 + d
            cpr = pltpu.make_async_copy(
                tab_hbm.at[t, d, pl.ds(0, MAIN)], row_v.at[pl.ds(0, MAIN)],
                semr)
            cpr.start()
            rbase = r * TAIL
            for h in range(0, TAIL, L):
                tvals = plsc.load_gather(
                    tail_v, [rbase + h + lax.iota(jnp.int32, L)])
                plsc.store_scatter(
                    row_v, [MAIN + h + lax.iota(jnp.int32, L)], tvals)
            half = lax.rem(seq, 2) * BATCH
            ibase = (r // DIM - f0) * BATCH
            cpr.wait()
            @pl.when(seq > 0)
            def _():
                pltpu.make_async_copy(
                    out_v.at[pl.ds(0, BATCH)],
                    out_hbm.at[t, d, :], semo).wait()

            extract(ibase, half)
            pltpu.make_async_copy(
                out_v.at[pl.ds(half, BATCH)], out_hbm.at[t, d, :], semo
            ).start()
            return seq + 1

        seq0 = lax.fori_loop(t_lo, t_hi, do_row, seq0)
    pltpu.make_async_copy(
        out_v.at[pl.ds(0, BATCH)], out_hbm.at[0, 0, :], semo).wait()


@jax.jit
def _embed_t(idx_t, tab3, tail1):
    mesh = plsc.VectorSubcoreMesh(
        core_axis_name="c", subcore_axis_name="s", num_cores=NC, num_subcores=NS
    )
    return pl.kernel(
        _row_body,
        out_type=jax.ShapeDtypeStruct((TR, 8, BATCH), jnp.float32),
        mesh=mesh,
        scratch_types=[
            pltpu.VMEM((2 * BATCH,), jnp.int32),
            pltpu.VMEM((VOCAB,), jnp.float32),
            pltpu.VMEM((R * TAIL,), jnp.float32),
            pltpu.VMEM((2 * BATCH,), jnp.float32),
            pltpu.SemaphoreType.DMA,
            pltpu.SemaphoreType.DMA,
        ],
        compiler_params=pltpu.CompilerParams(
            use_tc_tiling_on_sc=True, needs_layout_passes=False
        ),
    )(idx_t, tab3, tail1)


def kernel(indices, tables):
    idx_t = indices.T
    tab3 = jnp.transpose(tables, (0, 2, 1)).reshape(TR, 8, VOCAB)
    tail1 = jnp.transpose(tables[:, MAIN:, :], (0, 2, 1)).reshape(R * TAIL)
    out_t = _embed_t(idx_t, tab3, tail1)
    return out_t.reshape(R, BATCH).T


# speedup vs baseline: 1.1328x; 1.0076x over previous
import jax
import jax.numpy as jnp
from jax import lax
from jax.experimental import pallas as pl
from jax.experimental.pallas import tpu as pltpu
from jax.experimental.pallas import tpu_sc as plsc

NUM_FIELDS = 26
VOCAB = 100000
DIM = 16
BATCH = 4096

NC = 2
NS = 16
L = 16
NW = NC * NS

R = NUM_FIELDS * DIM
TR = R // 8
R_W = R // NW
MAIN = 99968
TAIL = VOCAB - MAIN
GROUPS = BATCH // L


def _row_body(idx_hbm, tab_hbm, tail_hbm, out_hbm,
              idx_v, row_v, out_v, semr, semo):
    wid = lax.axis_index("s") * NC + lax.axis_index("c")
    lo_row = wid * R_W
    f0 = lo_row // DIM
    f1 = jnp.minimum(f0 + 1, NUM_FIELDS - 1)
    pltpu.sync_copy(idx_hbm.at[f0, :], idx_v.at[pl.ds(0, BATCH)])
    pltpu.sync_copy(idx_hbm.at[f1, :], idx_v.at[pl.ds(BATCH, BATCH)])

    def extract(ibase, obase):
        def do_group(g, carry2):
            iv = idx_v[pl.ds(ibase + g * L, L)]
            out_v[pl.ds(obase + g * L, L)] = plsc.load_gather(row_v, [iv])
            return carry2

        lax.fori_loop(0, GROUPS, do_group, 0, unroll=8)

    seq0 = 0
    for d in range(8):
        t_lo = (lo_row + 7 - d) // 8
        t_hi = (lo_row + R_W + 7 - d) // 8

        def do_row(t, seq, d=d):
            r = t * 8 + d
            cpr = pltpu.make_async_copy(
                tab_hbm.at[t, d, pl.ds(0, MAIN)], row_v.at[pl.ds(0, MAIN)],
                semr)
            cpr.start()
            pltpu.sync_copy(tail_hbm.at[pl.ds(r * TAIL, TAIL)],
                            row_v.at[pl.ds(MAIN, TAIL)])
            half = lax.rem(seq, 2) * BATCH
            ibase = (r // DIM - f0) * BATCH
            cpr.wait()
            @pl.when(seq > 0)
            def _():
                pltpu.make_async_copy(
                    out_v.at[pl.ds(0, BATCH)],
                    out_hbm.at[t, d, :], semo).wait()

            extract(ibase, half)
            pltpu.make_async_copy(
                out_v.at[pl.ds(half, BATCH)], out_hbm.at[t, d, :], semo
            ).start()
            return seq + 1

        seq0 = lax.fori_loop(t_lo, t_hi, do_row, seq0)
    pltpu.make_async_copy(
        out_v.at[pl.ds(0, BATCH)], out_hbm.at[0, 0, :], semo).wait()


@jax.jit
def _embed_t(idx_t, tab3, tail1):
    mesh = plsc.VectorSubcoreMesh(
        core_axis_name="c", subcore_axis_name="s", num_cores=NC, num_subcores=NS
    )
    return pl.kernel(
        _row_body,
        out_type=jax.ShapeDtypeStruct((TR, 8, BATCH), jnp.float32),
        mesh=mesh,
        scratch_types=[
            pltpu.VMEM((2 * BATCH,), jnp.int32),
            pltpu.VMEM((VOCAB,), jnp.float32),
            pltpu.VMEM((2 * BATCH,), jnp.float32),
            pltpu.SemaphoreType.DMA,
            pltpu.SemaphoreType.DMA,
        ],
        compiler_params=pltpu.CompilerParams(
            use_tc_tiling_on_sc=True, needs_layout_passes=False
        ),
    )(idx_t, tab3, tail1)


def kernel(indices, tables):
    idx_t = indices.T
    tab3 = jnp.transpose(tables, (0, 2, 1)).reshape(TR, 8, VOCAB)
    tail1 = jnp.transpose(tables[:, MAIN:, :], (0, 2, 1)).reshape(R * TAIL)
    out_t = _embed_t(idx_t, tab3, tail1)
    return out_t.reshape(R, BATCH).T
